# Initial kernel scaffold; baseline (speedup 1.0000x reference)
import functools

import jax
import jax.numpy as jnp
import numpy as np
from jax import lax
from jax.experimental import pallas as pl
from jax.experimental.pallas import tpu as pltpu
from jax.experimental.pallas import tpu_sc as plsc

_N_LEVELS = 16
_F = 2
_LOG2_T = 19
_T = 1 << _LOG2_T
_MASK = _T - 1
_BASE_RES = 16
_SCALE = 1.4472692012786865
_P1 = np.int32(np.uint32(2654435761))
_P2 = np.int32(np.uint32(805459861))

_NC = 2
_NS = 16
_NW = _NC * _NS
_L = 16

_RES = [int(np.floor(_BASE_RES * (_SCALE ** l))) for l in range(_N_LEVELS)]
_DENSE = [(r + 1) ** 3 <= _T for r in _RES]

_C = 256
_BLK = _C // _L


def _body(n_points, x_hbm, table_hbm, out_hbm, xv, idxv, rowsv, outv, gsem):
    wid = lax.axis_index("s") * _NC + lax.axis_index("c")
    npw = n_points // _NW
    nchunks = npw // _C

    iota = lax.iota(jnp.int32, _L)
    zero16 = jnp.zeros((_L,), jnp.int32)
    one16 = jnp.full((_L,), 1, jnp.int32)
    two16 = jnp.full((_L,), 2, jnp.int32)

    @pl.loop(0, nchunks)
    def _chunk(ci):
        base = wid * npw + ci * _C
        pltpu.sync_copy(x_hbm.at[pl.ds(base, _C)], xv)

        @pl.loop(0, _BLK)
        def _block(b):
            j0 = b * _L
            jv = j0 + iota
            x0 = plsc.load_gather(xv, [jv, zero16])
            x1 = plsc.load_gather(xv, [jv, one16])
            x2 = plsc.load_gather(xv, [jv, two16])

            copies = []
            for l in range(_N_LEVELS):
                res = _RES[l]
                rf = float(res)
                s0 = x0 * rf
                s1 = x1 * rf
                s2 = x2 * rf
                b0 = s0.astype(jnp.int32)
                b1 = s1.astype(jnp.int32)
                b2 = s2.astype(jnp.int32)
                f0 = s0 - b0.astype(jnp.float32)
                f1 = s1 - b1.astype(jnp.float32)
                f2 = s2 - b2.astype(jnp.float32)

                off = l * (8 * _L)
                lo = l * _T
                if _DENSE[l]:
                    st = res + 1
                    a0 = (b0 + b1 * st + b2 * (st * st)) + lo
                    for c in range(8):
                        i, j, k = c >> 2, (c >> 1) & 1, c & 1
                        cst = i + j * st + k * st * st
                        idxv[pl.ds(off + c * _L, _L)] = a0 + cst
                else:
                    v0 = b1 * _P1
                    v1 = v0 + _P1
                    w0 = b2 * _P2
                    w1 = w0 + _P2
                    bx = (b0, b0 + 1)
                    vv = (v0, v1)
                    ww = (w0, w1)
                    xu = [bx[i] ^ vv[j] for i in range(2) for j in range(2)]
                    for c in range(8):
                        i, j, k = c >> 2, (c >> 1) & 1, c & 1
                        h = ((xu[i * 2 + j] ^ ww[k]) & _MASK) + lo
                        idxv[pl.ds(off + c * _L, _L)] = h
                cp = pltpu.async_copy(
                    table_hbm.at[idxv.at[pl.ds(off, 8 * _L)]], rowsv.at[l], gsem
                )
                copies.append((cp, f0, f1, f2))

            rowj = jv
            for l in range(_N_LEVELS):
                cp, f0, f1, f2 = copies[l]
                cp.wait()
                g0 = 1.0 - f0
                g1 = 1.0 - f1
                g2 = 1.0 - f2
                tx = (g0, f0)
                ty = (g1, f1)
                tz = (g2, f2)
                wxy = [tx[i] * ty[j] for i in range(2) for j in range(2)]
                lvec = jnp.full((_L,), l, jnp.int32)
                acc0 = None
                acc1 = None
                for c in range(8):
                    i, j, k = c >> 2, (c >> 1) & 1, c & 1
                    w = wxy[i * 2 + j] * tz[k]
                    pos = c * _L + iota
                    ft0 = plsc.load_gather(rowsv, [lvec, pos, zero16])
                    ft1 = plsc.load_gather(rowsv, [lvec, pos, one16])
                    if acc0 is None:
                        acc0 = w * ft0
                        acc1 = w * ft1
                    else:
                        acc0 = acc0 + w * ft0
                        acc1 = acc1 + w * ft1
                plsc.store_scatter(outv, [rowj, jnp.full((_L,), 2 * l, jnp.int32)], acc0)
                plsc.store_scatter(outv, [rowj, jnp.full((_L,), 2 * l + 1, jnp.int32)], acc1)

        pltpu.sync_copy(outv, out_hbm.at[pl.ds(base, _C)])


@jax.jit
def _hashgrid(x, table):
    n = x.shape[0]
    mesh = plsc.VectorSubcoreMesh(core_axis_name="c", subcore_axis_name="s")
    fn = pl.kernel(
        functools.partial(_body, n),
        out_type=jax.ShapeDtypeStruct((n, 2 * _N_LEVELS), jnp.float32),
        mesh=mesh,
        scratch_types=[
            pltpu.VMEM((_C, 3), jnp.float32),
            pltpu.VMEM((_N_LEVELS * 8 * _L,), jnp.int32),
            pltpu.VMEM((_N_LEVELS, 8 * _L, _F), jnp.float32),
            pltpu.VMEM((_C, 2 * _N_LEVELS), jnp.float32),
            pltpu.SemaphoreType.DMA,
        ],
    )
    return fn(x, table)


def kernel(x, table):
    return _hashgrid(x, table)


# speedup vs baseline: 2.1078x; 2.1078x over previous
import functools

import jax
import jax.numpy as jnp
import numpy as np
from jax import lax
from jax.experimental import pallas as pl
from jax.experimental.pallas import tpu as pltpu
from jax.experimental.pallas import tpu_sc as plsc

_N_LEVELS = 16
_F = 2
_LOG2_T = 19
_T = 1 << _LOG2_T
_MASK = _T - 1
_BASE_RES = 16
_SCALE = 1.4472692012786865
_P1 = np.int32(np.uint32(2654435761))
_P2 = np.int32(np.uint32(805459861))

_NC = 2
_NS = 16
_NW = _NC * _NS
_L = 16

_RES = [int(np.floor(_BASE_RES * (_SCALE ** l))) for l in range(_N_LEVELS)]
_DENSE = [(r + 1) ** 3 <= _T for r in _RES]

_C = 256
_BLK = _C // _L
_LW = _F * 8 * _L


def _body(n_points, x_hbm, table_hbm, out_hbm, xv, idxv, rowsv, outv, gsem):
    wid = lax.axis_index("s") * _NC + lax.axis_index("c")
    npw = n_points // _NW
    nchunks = npw // _C

    iota = lax.iota(jnp.int32, _L)
    iota3 = iota * 3

    @pl.loop(0, nchunks)
    def _chunk(ci):
        base = wid * npw + ci * _C
        pltpu.sync_copy(x_hbm.at[pl.ds(base * 3, _C * 3)], xv)

        @pl.loop(0, _BLK)
        def _block(b):
            j0 = b * _L
            jv3 = j0 * 3 + iota3
            x0 = plsc.load_gather(xv, [jv3])
            x1 = plsc.load_gather(xv, [jv3 + 1])
            x2 = plsc.load_gather(xv, [jv3 + 2])

            copies = []
            for l in range(_N_LEVELS):
                res = _RES[l]
                rf = float(res)
                s0 = x0 * rf
                s1 = x1 * rf
                s2 = x2 * rf
                b0 = s0.astype(jnp.int32)
                b1 = s1.astype(jnp.int32)
                b2 = s2.astype(jnp.int32)
                f0 = s0 - b0.astype(jnp.float32)
                f1 = s1 - b1.astype(jnp.float32)
                f2 = s2 - b2.astype(jnp.float32)

                off = l * _LW
                lo = l * _T
                hs = []
                if _DENSE[l]:
                    st = res + 1
                    a0 = (b0 + b1 * st + b2 * (st * st)) + lo
                    for c in range(8):
                        i, j, k = c >> 2, (c >> 1) & 1, c & 1
                        hs.append(a0 + (i + j * st + k * st * st))
                else:
                    v0 = b1 * _P1
                    v1 = v0 + _P1
                    w0 = b2 * _P2
                    w1 = w0 + _P2
                    bx = (b0, b0 + 1)
                    vv = (v0, v1)
                    ww = (w0, w1)
                    xu = [bx[i] ^ vv[j] for i in range(2) for j in range(2)]
                    for c in range(8):
                        i, j, k = c >> 2, (c >> 1) & 1, c & 1
                        hs.append((((xu[i * 2 + j] ^ ww[k]) & _MASK) + lo))
                for c in range(8):
                    h2 = hs[c] + hs[c]
                    idxv[pl.ds(off + c * 32, _L)] = h2
                    idxv[pl.ds(off + c * 32 + _L, _L)] = h2 + 1
                cp0 = pltpu.async_copy(
                    table_hbm.at[idxv.at[pl.ds(off, 128)]],
                    rowsv.at[pl.ds(off, 128)], gsem,
                )
                cp1 = pltpu.async_copy(
                    table_hbm.at[idxv.at[pl.ds(off + 128, 128)]],
                    rowsv.at[pl.ds(off + 128, 128)], gsem,
                )
                copies.append((cp0, cp1, f0, f1, f2))

            ob = j0 * 32 + iota * 32
            for l in range(_N_LEVELS):
                cp0, cp1, f0, f1, f2 = copies[l]
                cp0.wait()
                cp1.wait()
                g0 = 1.0 - f0
                g1 = 1.0 - f1
                g2 = 1.0 - f2
                tx = (g0, f0)
                ty = (g1, f1)
                tz = (g2, f2)
                wxy = [tx[i] * ty[j] for i in range(2) for j in range(2)]
                off = l * _LW
                acc0 = None
                acc1 = None
                for c in range(8):
                    i, j, k = c >> 2, (c >> 1) & 1, c & 1
                    w = wxy[i * 2 + j] * tz[k]
                    ft0 = rowsv[pl.ds(off + c * 32, _L)]
                    ft1 = rowsv[pl.ds(off + c * 32 + _L, _L)]
                    if acc0 is None:
                        acc0 = w * ft0
                        acc1 = w * ft1
                    else:
                        acc0 = acc0 + w * ft0
                        acc1 = acc1 + w * ft1
                plsc.store_scatter(outv, [ob + 2 * l], acc0)
                plsc.store_scatter(outv, [ob + (2 * l + 1)], acc1)

        pltpu.sync_copy(outv, out_hbm.at[pl.ds(base * 32, _C * 32)])


@jax.jit
def _hashgrid(x, table):
    n = x.shape[0]
    mesh = plsc.VectorSubcoreMesh(core_axis_name="c", subcore_axis_name="s")
    fn = pl.kernel(
        functools.partial(_body, n),
        out_type=jax.ShapeDtypeStruct((n * 2 * _N_LEVELS,), jnp.float32),
        mesh=mesh,
        compiler_params=pltpu.CompilerParams(needs_layout_passes=False),
        scratch_types=[
            pltpu.VMEM((_C * 3,), jnp.float32),
            pltpu.VMEM((_N_LEVELS * _LW,), jnp.int32),
            pltpu.VMEM((_N_LEVELS * _LW,), jnp.float32),
            pltpu.VMEM((_C * 2 * _N_LEVELS,), jnp.float32),
            pltpu.SemaphoreType.DMA,
        ],
    )
    out = fn(x.reshape(-1), table.reshape(-1))
    return out.reshape(n, 2 * _N_LEVELS)


def kernel(x, table):
    return _hashgrid(x, table)
